# Initial kernel scaffold; baseline (speedup 1.0000x reference)
import jax
import jax.numpy as jnp
from jax.experimental import pallas as pl


def kernel(query, image):
    raise NotImplementedError("write your pallas kernel here")


# speedup vs baseline: 8.5645x; 8.5645x over previous
import functools

import jax
import jax.numpy as jnp
from jax import lax
from jax.experimental import pallas as pl
from jax.experimental.pallas import tpu as pltpu
from jax.experimental.pallas import tpu_sc as plsc

_TOPW = 4
_CHUNK = 128
_LANES = 16
import numpy as np

_NEG = np.float32(-3.0e38)
_BIGI = np.int32(1 << 30)


def _score_body(q_ref, img_ref, s_ref, cmax_ref):
    q = q_ref[0]
    img = img_ref[0]
    s = lax.dot_general(q, img, (((1,), (1,)), ((), ())),
                        preferred_element_type=jnp.float32)
    s_ref[0] = s
    nq, n = s.shape
    cmax_ref[0] = jnp.max(s.reshape(nq, n // _CHUNK, _CHUNK), axis=-1)


@functools.lru_cache(maxsize=None)
def _tc_scores_fn(nb, nq, n, d):
    return pl.pallas_call(
        _score_body,
        grid=(nb,),
        in_specs=[
            pl.BlockSpec((1, nq, d), lambda i: (i, 0, 0)),
            pl.BlockSpec((1, n, d), lambda i: (i, 0, 0)),
        ],
        out_specs=[
            pl.BlockSpec((1, nq, n), lambda i: (i, 0, 0)),
            pl.BlockSpec((1, nq, n // _CHUNK), lambda i: (i, 0, 0)),
        ],
        out_shape=[
            jax.ShapeDtypeStruct((nb, nq, n), jnp.float32),
            jax.ShapeDtypeStruct((nb, nq, n // _CHUNK), jnp.float32),
        ],
    )


@functools.lru_cache(maxsize=None)
def _sc_route_fn(nb, nq, n):
    nchunks = n // _CHUNK
    ncand = _TOPW * _CHUNK
    out_len = nq * _TOPW
    assert out_len == _LANES
    mesh = plsc.VectorSubcoreMesh(core_axis_name="c", subcore_axis_name="s",
                                  num_cores=2, num_subcores=16)

    @functools.partial(
        pl.kernel,
        mesh=mesh,
        compiler_params=pltpu.CompilerParams(needs_layout_passes=False),
        out_type=jax.ShapeDtypeStruct((nb * out_len,), jnp.int32),
        scratch_types=[
            pltpu.VMEM((nchunks,), jnp.float32),
            pltpu.VMEM((ncand,), jnp.float32),
            pltpu.VMEM((ncand,), jnp.int32),
            pltpu.VMEM((_LANES,), jnp.int32),
        ],
    )
    def sc_route(cmax_hbm, scores_hbm, out_hbm, cmax_v, cand_v, cidx_v, out_v):
        b = lax.axis_index("s") * 2 + lax.axis_index("c")
        lane = lax.broadcasted_iota(jnp.int32, (_LANES,), 0)
        out_acc = jnp.zeros((_LANES,), jnp.int32)

        for q in range(nq):
            r = b * nq + q
            pltpu.sync_copy(cmax_hbm.at[pl.ds(r * nchunks, nchunks)], cmax_v)

            chosen_chunks = []
            for p in range(_TOPW):
                excl = list(chosen_chunks)

                def _maxbody(j, m, excl=excl):
                    v = cmax_v[pl.ds(j * _LANES, _LANES)]
                    idxv = lane + j * _LANES
                    for cc in excl:
                        v = jnp.where(idxv == cc, _NEG, v)
                    return jnp.maximum(m, v)

                m = lax.fori_loop(0, nchunks // _LANES, _maxbody,
                                  jnp.full((_LANES,), _NEG, jnp.float32))
                gmax = jnp.max(m)

                def _argbody(j, best, excl=excl, gmax=gmax):
                    v = cmax_v[pl.ds(j * _LANES, _LANES)]
                    idxv = lane + j * _LANES
                    for cc in excl:
                        v = jnp.where(idxv == cc, _NEG, v)
                    c = jnp.where(v == gmax, idxv, _BIGI)
                    return jnp.minimum(best, jnp.min(c))

                best = lax.fori_loop(0, nchunks // _LANES, _argbody, _BIGI)
                chosen_chunks.append(best)

            for p, cc in enumerate(chosen_chunks):
                base = cc * _CHUNK
                pltpu.sync_copy(
                    scores_hbm.at[pl.ds(r * n + base, _CHUNK)],
                    cand_v.at[pl.ds(p * _CHUNK, _CHUNK)],
                )
                for t in range(_CHUNK // _LANES):
                    cidx_v[pl.ds(p * _CHUNK + t * _LANES, _LANES)] = (
                        base + t * _LANES + lane)

            chosen_idx = []
            for p in range(_TOPW):
                excl = list(chosen_idx)

                def _cmaxbody(j, m, excl=excl):
                    v = cand_v[pl.ds(j * _LANES, _LANES)]
                    gidx = cidx_v[pl.ds(j * _LANES, _LANES)]
                    for ci in excl:
                        v = jnp.where(gidx == ci, _NEG, v)
                    return jnp.maximum(m, v)

                m = lax.fori_loop(0, ncand // _LANES, _cmaxbody,
                                  jnp.full((_LANES,), _NEG, jnp.float32))
                gmax = jnp.max(m)

                def _cargbody(j, best, excl=excl, gmax=gmax):
                    v = cand_v[pl.ds(j * _LANES, _LANES)]
                    gidx = cidx_v[pl.ds(j * _LANES, _LANES)]
                    for ci in excl:
                        v = jnp.where(gidx == ci, _NEG, v)
                    c = jnp.where(v == gmax, gidx, _BIGI)
                    return jnp.minimum(best, jnp.min(c))

                best = lax.fori_loop(0, ncand // _LANES, _cargbody, _BIGI)
                chosen_idx.append(best)
                out_acc = jnp.where(lane == q * _TOPW + p, best, out_acc)

        out_v[...] = out_acc
        pltpu.sync_copy(out_v, out_hbm.at[pl.ds(b * out_len, out_len)])

    return sc_route


def kernel(query, image):
    nb, nq, d = query.shape
    n = image.shape[1]
    scores, cmax = _tc_scores_fn(nb, nq, n, d)(query, image)
    out = _sc_route_fn(nb, nq, n)(cmax.reshape(-1), scores.reshape(-1))
    return out.reshape(nb, nq, _TOPW)


# speedup vs baseline: 9.0492x; 1.0566x over previous
import functools

import jax
import jax.numpy as jnp
import numpy as np
from jax import lax
from jax.experimental import pallas as pl
from jax.experimental.pallas import tpu as pltpu
from jax.experimental.pallas import tpu_sc as plsc

_TOPW = 4
_CHUNK = 128
_LANES = 16
_NEG = np.float32(-3.0e38)
_BIGI = np.int32(1 << 30)


def _score_body(q_ref, img_ref, cand_ref, cid_ref, s_scratch):
    q = q_ref[0]
    img = img_ref[0]
    s = lax.dot_general(q, img, (((1,), (1,)), ((), ())),
                        preferred_element_type=jnp.float32)
    nq, n = s.shape
    s_scratch[...] = s
    nchunks = n // _CHUNK
    cmax = jnp.max(s.reshape(nq, nchunks, _CHUNK), axis=-1)
    cidx = lax.broadcasted_iota(jnp.int32, (nq, nchunks), 1)
    rowi = lax.broadcasted_iota(jnp.int32, (nq, 1), 0)

    sels = []
    work = cmax
    for p in range(_TOPW):
        m = jnp.max(work, axis=1, keepdims=True)
        sel = jnp.min(jnp.where(work == m, cidx, _BIGI), axis=1,
                      keepdims=True)
        work = jnp.where(cidx == sel, _NEG, work)
        for qi in range(nq):
            sc = jnp.max(jnp.where(rowi == qi, sel, jnp.int32(-1)))
            sels.append((qi, p, sc))

    r2 = lax.broadcasted_iota(jnp.int32, (nq, _LANES), 0)
    c2 = lax.broadcasted_iota(jnp.int32, (nq, _LANES), 1)
    ids_arr = jnp.zeros((nq, _LANES), jnp.int32)
    for qi, p, sc in sels:
        ids_arr = jnp.where((r2 == qi) & (c2 == p), sc, ids_arr)
    cid_ref[0] = ids_arr

    for qi, p, sc in sels:
        start = pl.multiple_of(sc * _CHUNK, _CHUNK)
        cand_ref[0, pl.ds(qi, 1), pl.ds(p * _CHUNK, _CHUNK)] = (
            s_scratch[pl.ds(qi, 1), pl.ds(start, _CHUNK)])


@functools.lru_cache(maxsize=None)
def _tc_scores_fn(nb, nq, n, d):
    ncand = _TOPW * _CHUNK
    return pl.pallas_call(
        _score_body,
        grid=(nb,),
        in_specs=[
            pl.BlockSpec((1, nq, d), lambda i: (i, 0, 0)),
            pl.BlockSpec((1, n, d), lambda i: (i, 0, 0)),
        ],
        out_specs=[
            pl.BlockSpec((1, nq, ncand), lambda i: (i, 0, 0)),
            pl.BlockSpec((1, nq, _LANES), lambda i: (i, 0, 0)),
        ],
        out_shape=[
            jax.ShapeDtypeStruct((nb, nq, ncand), jnp.float32),
            jax.ShapeDtypeStruct((nb, nq, _LANES), jnp.int32),
        ],
        scratch_shapes=[pltpu.VMEM((nq, n), jnp.float32)],
    )


@functools.lru_cache(maxsize=None)
def _sc_route_fn(nb, nq, n):
    ncand = _TOPW * _CHUNK
    out_len = nq * _TOPW
    assert out_len == _LANES
    mesh = plsc.VectorSubcoreMesh(core_axis_name="c", subcore_axis_name="s",
                                  num_cores=2, num_subcores=16)

    @functools.partial(
        pl.kernel,
        mesh=mesh,
        compiler_params=pltpu.CompilerParams(needs_layout_passes=False),
        out_type=jax.ShapeDtypeStruct((nb * out_len,), jnp.int32),
        scratch_types=[
            pltpu.VMEM((ncand,), jnp.float32),
            pltpu.VMEM((ncand,), jnp.int32),
            pltpu.VMEM((_LANES,), jnp.int32),
            pltpu.VMEM((_LANES,), jnp.int32),
        ],
    )
    def sc_route(cand_hbm, cid_hbm, out_hbm, cand_v, cidx_v, ids_v, out_v):
        b = lax.axis_index("s") * 2 + lax.axis_index("c")
        lane = lax.broadcasted_iota(jnp.int32, (_LANES,), 0)
        out_acc = jnp.zeros((_LANES,), jnp.int32)

        for q in range(nq):
            r = b * nq + q
            pltpu.sync_copy(cand_hbm.at[pl.ds(r * ncand, ncand)], cand_v)
            pltpu.sync_copy(cid_hbm.at[pl.ds(r * _LANES, _LANES)], ids_v)
            idv = ids_v[...]

            for p in range(_TOPW):
                base = idv[p] * _CHUNK
                for t in range(_CHUNK // _LANES):
                    cidx_v[pl.ds(p * _CHUNK + t * _LANES, _LANES)] = (
                        base + t * _LANES + lane)

            chosen_idx = []
            for p in range(_TOPW):
                excl = list(chosen_idx)

                def _cmaxbody(j, m, excl=excl):
                    v = cand_v[pl.ds(j * _LANES, _LANES)]
                    gidx = cidx_v[pl.ds(j * _LANES, _LANES)]
                    for ci in excl:
                        v = jnp.where(gidx == ci, _NEG, v)
                    return jnp.maximum(m, v)

                m = lax.fori_loop(0, ncand // _LANES, _cmaxbody,
                                  jnp.full((_LANES,), _NEG, jnp.float32))
                gmax = jnp.max(m)

                def _cargbody(j, best, excl=excl, gmax=gmax):
                    v = cand_v[pl.ds(j * _LANES, _LANES)]
                    gidx = cidx_v[pl.ds(j * _LANES, _LANES)]
                    for ci in excl:
                        v = jnp.where(gidx == ci, _NEG, v)
                    c = jnp.where(v == gmax, gidx, _BIGI)
                    return jnp.minimum(best, jnp.min(c))

                best = lax.fori_loop(0, ncand // _LANES, _cargbody, _BIGI)
                chosen_idx.append(best)
                out_acc = jnp.where(lane == q * _TOPW + p, best, out_acc)

        out_v[...] = out_acc
        pltpu.sync_copy(out_v, out_hbm.at[pl.ds(b * out_len, out_len)])

    return sc_route


def kernel(query, image):
    nb, nq, d = query.shape
    n = image.shape[1]
    cand, cid = _tc_scores_fn(nb, nq, n, d)(query, image)
    out = _sc_route_fn(nb, nq, n)(cand.reshape(-1), cid.reshape(-1))
    return out.reshape(nb, nq, _TOPW)


# speedup vs baseline: 9.1282x; 1.0087x over previous
import functools

import jax
import jax.numpy as jnp
import numpy as np
from jax import lax
from jax.experimental import pallas as pl
from jax.experimental.pallas import tpu as pltpu
from jax.experimental.pallas import tpu_sc as plsc

_TOPW = 4
_CHUNK = 128
_LANES = 16
_NEG = np.float32(-3.0e38)
_BIGI = np.int32(1 << 30)


def _score_body(q_ref, img_ref, cand_ref, cid_ref, s_scratch):
    q = q_ref[0]
    img = img_ref[0]
    s = lax.dot_general(q, img, (((1,), (1,)), ((), ())),
                        preferred_element_type=jnp.float32)
    nq, n = s.shape
    s_scratch[...] = s
    nchunks = n // _CHUNK
    cmax = jnp.max(s.reshape(nq, nchunks, _CHUNK), axis=-1)
    cidx = lax.broadcasted_iota(jnp.int32, (nq, nchunks), 1)
    rowi = lax.broadcasted_iota(jnp.int32, (nq, 1), 0)

    sels = []
    work = cmax
    for p in range(_TOPW):
        m = jnp.max(work, axis=1, keepdims=True)
        sel = jnp.min(jnp.where(work == m, cidx, _BIGI), axis=1,
                      keepdims=True)
        work = jnp.where(cidx == sel, _NEG, work)
        for qi in range(nq):
            sc = jnp.max(jnp.where(rowi == qi, sel, jnp.int32(-1)))
            sels.append((qi, p, sc))

    r2 = lax.broadcasted_iota(jnp.int32, (nq, _LANES), 0)
    c2 = lax.broadcasted_iota(jnp.int32, (nq, _LANES), 1)
    ids_arr = jnp.zeros((nq, _LANES), jnp.int32)
    for qi, p, sc in sels:
        ids_arr = jnp.where((r2 == qi) & (c2 == p), sc, ids_arr)
    cid_ref[0] = ids_arr

    for qi, p, sc in sels:
        start = pl.multiple_of(sc * _CHUNK, _CHUNK)
        cand_ref[0, pl.ds(qi, 1), pl.ds(p * _CHUNK, _CHUNK)] = (
            s_scratch[pl.ds(qi, 1), pl.ds(start, _CHUNK)])


@functools.lru_cache(maxsize=None)
def _tc_scores_fn(nb, nq, n, d):
    ncand = _TOPW * _CHUNK
    return pl.pallas_call(
        _score_body,
        grid=(nb,),
        in_specs=[
            pl.BlockSpec((1, nq, d), lambda i: (i, 0, 0)),
            pl.BlockSpec((1, n, d), lambda i: (i, 0, 0)),
        ],
        out_specs=[
            pl.BlockSpec((1, nq, ncand), lambda i: (i, 0, 0)),
            pl.BlockSpec((1, nq, _LANES), lambda i: (i, 0, 0)),
        ],
        out_shape=[
            jax.ShapeDtypeStruct((nb, nq, ncand), jnp.float32),
            jax.ShapeDtypeStruct((nb, nq, _LANES), jnp.int32),
        ],
        scratch_shapes=[pltpu.VMEM((nq, n), jnp.float32)],
    )


@functools.lru_cache(maxsize=None)
def _sc_route_fn(nb, nq, n):
    ncand = _TOPW * _CHUNK
    out_len = nq * _TOPW
    assert out_len == _LANES
    mesh = plsc.VectorSubcoreMesh(core_axis_name="c", subcore_axis_name="s",
                                  num_cores=2, num_subcores=16)

    @functools.partial(
        pl.kernel,
        mesh=mesh,
        compiler_params=pltpu.CompilerParams(needs_layout_passes=False),
        out_type=jax.ShapeDtypeStruct((nb * out_len,), jnp.int32),
        scratch_types=[
            pltpu.VMEM((nq, ncand), jnp.float32),
            pltpu.VMEM((ncand,), jnp.int32),
            pltpu.VMEM((nq, _LANES), jnp.int32),
            pltpu.VMEM((_LANES,), jnp.int32),
            pltpu.SemaphoreType.DMA,
        ],
    )
    def sc_route(cand_hbm, cid_hbm, out_hbm, cand_vv, cidx_v, ids_vv, out_v,
                 sem):
        b = lax.axis_index("s") * 2 + lax.axis_index("c")
        lane = lax.broadcasted_iota(jnp.int32, (_LANES,), 0)
        out_acc = jnp.zeros((_LANES,), jnp.int32)

        handles = []
        for q in range(nq):
            r = b * nq + q
            handles.append(pltpu.async_copy(
                cand_hbm.at[pl.ds(r, 1)], cand_vv.at[pl.ds(q, 1)], sem))
            handles.append(pltpu.async_copy(
                cid_hbm.at[pl.ds(r, 1)], ids_vv.at[pl.ds(q, 1)], sem))
        for h in handles:
            h.wait()

        for q in range(nq):
            idv = ids_vv[q]

            for p in range(_TOPW):
                base = idv[p] * _CHUNK
                for t in range(_CHUNK // _LANES):
                    cidx_v[pl.ds(p * _CHUNK + t * _LANES, _LANES)] = (
                        base + t * _LANES + lane)

            chosen_idx = []
            for p in range(_TOPW):
                excl = list(chosen_idx)

                def _cmaxbody(j, m, excl=excl, q=q):
                    v = cand_vv[q, pl.ds(j * _LANES, _LANES)]
                    gidx = cidx_v[pl.ds(j * _LANES, _LANES)]
                    for ci in excl:
                        v = jnp.where(gidx == ci, _NEG, v)
                    return jnp.maximum(m, v)

                m = lax.fori_loop(0, ncand // _LANES, _cmaxbody,
                                  jnp.full((_LANES,), _NEG, jnp.float32))
                gmax = jnp.max(m)

                def _cargbody(j, best, excl=excl, gmax=gmax, q=q):
                    v = cand_vv[q, pl.ds(j * _LANES, _LANES)]
                    gidx = cidx_v[pl.ds(j * _LANES, _LANES)]
                    for ci in excl:
                        v = jnp.where(gidx == ci, _NEG, v)
                    c = jnp.where(v == gmax, gidx, _BIGI)
                    return jnp.minimum(best, jnp.min(c))

                best = lax.fori_loop(0, ncand // _LANES, _cargbody, _BIGI)
                chosen_idx.append(best)
                out_acc = jnp.where(lane == q * _TOPW + p, best, out_acc)

        out_v[...] = out_acc
        pltpu.sync_copy(out_v, out_hbm.at[pl.ds(b * out_len, out_len)])

    return sc_route


def kernel(query, image):
    nb, nq, d = query.shape
    n = image.shape[1]
    cand, cid = _tc_scores_fn(nb, nq, n, d)(query, image)
    out = _sc_route_fn(nb, nq, n)(
        cand.reshape(nb * nq, _TOPW * _CHUNK), cid.reshape(nb * nq, _LANES))
    return out.reshape(nb, nq, _TOPW)
